# baseline (speedup 1.0000x reference)
import functools

import jax
import jax.numpy as jnp
from jax import lax
from jax.experimental import pallas as pl
from jax.experimental.pallas import tpu as pltpu
from jax.experimental.pallas import tpu_sc as plsc

N_PROPS = 528
D_HALF = 256
D_FULL = 512
PE_ROWS = 64
ROWS_PER_WORKER = 24
N_ACTIVE_WORKERS = N_PROPS // ROWS_PER_WORKER

_SC_MESH = plsc.VectorSubcoreMesh(core_axis_name="c", subcore_axis_name="s")


@functools.partial(
    pl.kernel,
    mesh=_SC_MESH,
    out_type=jax.ShapeDtypeStruct((N_PROPS, D_HALF), jnp.float32),
    scratch_types=[
        pltpu.VMEM((ROWS_PER_WORKER,), jnp.int32),
        pltpu.VMEM((ROWS_PER_WORKER, D_HALF), jnp.float32),
        pltpu.SemaphoreType.DMA,
    ],
)
def _sc_gather_bias(table_hbm, idx_hbm, out_hbm, idx_v, rows_v, sem):
    wid = lax.axis_index("s") * 2 + lax.axis_index("c")

    @pl.when(wid < N_ACTIVE_WORKERS)
    def _():
        base = wid * ROWS_PER_WORKER
        pltpu.sync_copy(idx_hbm.at[pl.ds(base, ROWS_PER_WORKER)], idx_v)
        pltpu.async_copy(table_hbm.at[idx_v], rows_v, sem).wait()
        pltpu.sync_copy(rows_v, out_hbm.at[pl.ds(base, ROWS_PER_WORKER)])


def _add_body(x_ref, b_ref, o_ref):
    o_ref[...] = x_ref[...] + b_ref[...][None, :, None, :]


def kernel(x, pe, props):
    bsz = x.shape[0]
    table = pe.reshape(PE_ROWS, D_HALF)
    idx = props[:, 0]

    bias = _sc_gather_bias(table, idx)

    b_blk = 8
    x4 = x.reshape(bsz, N_PROPS, 2, D_HALF)
    out = pl.pallas_call(
        _add_body,
        grid=(bsz // b_blk,),
        in_specs=[
            pl.BlockSpec((b_blk, N_PROPS, 2, D_HALF), lambda i: (i, 0, 0, 0)),
            pl.BlockSpec((N_PROPS, D_HALF), lambda i: (0, 0)),
        ],
        out_specs=pl.BlockSpec((b_blk, N_PROPS, 2, D_HALF), lambda i: (i, 0, 0, 0)),
        out_shape=jax.ShapeDtypeStruct((bsz, N_PROPS, 2, D_HALF), x.dtype),
        compiler_params=pltpu.CompilerParams(
            dimension_semantics=("parallel",),
        ),
    )(x4, bias)
    return out.reshape(x.shape)


# speedup vs baseline: 4.4339x; 4.4339x over previous
import functools

import jax
import jax.numpy as jnp
from jax import lax
from jax.experimental import pallas as pl
from jax.experimental.pallas import tpu as pltpu
from jax.experimental.pallas import tpu_sc as plsc

N_PROPS = 528
D_HALF = 256
D_FULL = 512
PE_ROWS = 64
ROWS_PER_WORKER = 24
N_ACTIVE_WORKERS = N_PROPS // ROWS_PER_WORKER

_SC_MESH = plsc.VectorSubcoreMesh(core_axis_name="c", subcore_axis_name="s")


@functools.partial(
    pl.kernel,
    mesh=_SC_MESH,
    out_type=jax.ShapeDtypeStruct((N_PROPS, D_HALF), jnp.float32),
    scratch_types=[
        pltpu.VMEM((ROWS_PER_WORKER,), jnp.int32),
        pltpu.VMEM((ROWS_PER_WORKER, D_HALF), jnp.float32),
        pltpu.SemaphoreType.DMA,
    ],
)
def _sc_gather_bias(table_hbm, idx_hbm, out_hbm, idx_v, rows_v, sem):
    wid = lax.axis_index("s") * 2 + lax.axis_index("c")

    @pl.when(wid < N_ACTIVE_WORKERS)
    def _():
        base = wid * ROWS_PER_WORKER
        pltpu.sync_copy(idx_hbm.at[pl.ds(base, ROWS_PER_WORKER)], idx_v)
        pltpu.async_copy(table_hbm.at[idx_v], rows_v, sem).wait()
        pltpu.sync_copy(rows_v, out_hbm.at[pl.ds(base, ROWS_PER_WORKER)])


def _add_body(x_ref, b_ref, o_ref):
    b = b_ref[...]
    o_ref[:, :, :D_HALF] = x_ref[:, :, :D_HALF] + b[None]
    o_ref[:, :, D_HALF:] = x_ref[:, :, D_HALF:] + b[None]


def kernel(x, pe, props):
    bsz = x.shape[0]
    table = pe.reshape(PE_ROWS, D_HALF)
    idx = props[:, 0]

    bias = _sc_gather_bias(table, idx)

    b_blk = 8
    out = pl.pallas_call(
        _add_body,
        grid=(bsz // b_blk,),
        in_specs=[
            pl.BlockSpec((b_blk, N_PROPS, D_FULL), lambda i: (i, 0, 0)),
            pl.BlockSpec((N_PROPS, D_HALF), lambda i: (0, 0)),
        ],
        out_specs=pl.BlockSpec((b_blk, N_PROPS, D_FULL), lambda i: (i, 0, 0)),
        out_shape=jax.ShapeDtypeStruct(x.shape, x.dtype),
        compiler_params=pltpu.CompilerParams(
            dimension_semantics=("parallel",),
        ),
    )(x, bias)
    return out


# speedup vs baseline: 4.9599x; 1.1186x over previous
import jax
import jax.numpy as jnp
from jax import lax
from jax.experimental import pallas as pl
from jax.experimental.pallas import tpu as pltpu

N_PROPS = 528
D_HALF = 256
D_FULL = 512
PE_ROWS = 64


def _add_body(x_ref, pe_ref, left_ref, o_ref, bias_ref):
    @pl.when(pl.program_id(0) == 0)
    def _():
        left = left_ref[...]
        iota = lax.broadcasted_iota(jnp.int32, (N_PROPS, PE_ROWS), 1)
        onehot = jnp.where(left == iota, 1.0, 0.0).astype(jnp.float32)
        bias_ref[...] = jnp.dot(onehot, pe_ref[...],
                                preferred_element_type=jnp.float32)

    b = bias_ref[...]
    o_ref[:, :, :D_HALF] = x_ref[:, :, :D_HALF] + b[None]
    o_ref[:, :, D_HALF:] = x_ref[:, :, D_HALF:] + b[None]


def kernel(x, pe, props):
    bsz = x.shape[0]
    table = pe.reshape(PE_ROWS, D_HALF)
    left = props[:, :1]

    b_blk = 8
    out = pl.pallas_call(
        _add_body,
        grid=(bsz // b_blk,),
        in_specs=[
            pl.BlockSpec((b_blk, N_PROPS, D_FULL), lambda i: (i, 0, 0)),
            pl.BlockSpec((PE_ROWS, D_HALF), lambda i: (0, 0)),
            pl.BlockSpec((N_PROPS, 1), lambda i: (0, 0)),
        ],
        out_specs=pl.BlockSpec((b_blk, N_PROPS, D_FULL), lambda i: (i, 0, 0)),
        out_shape=jax.ShapeDtypeStruct(x.shape, x.dtype),
        scratch_shapes=[pltpu.VMEM((N_PROPS, D_HALF), jnp.float32)],
        compiler_params=pltpu.CompilerParams(
            dimension_semantics=("arbitrary",),
        ),
    )(x, table, left)
    return out
